# baseline (speedup 1.0000x reference)
import jax
import jax.numpy as jnp
from jax.experimental import pallas as pl
from jax.experimental.pallas import tpu as pltpu


def _cell_kernel(h_ref, c_ref, hb_ref, ht_ref, z_ref, zb_ref,
                 W_ref, R_ref, U_ref, b_ref, wz_ref, rz_ref, uz_ref,
                 bz_ref,
                 hout_ref, cout_ref, zout_ref):
    BT = h_ref.shape[0]
    base = pl.program_id(0) * BT
    z = z_ref[pl.ds(base, BT), :]
    zb = zb_ref[pl.ds(base, BT), :]
    h = h_ref[...]
    hb = hb_ref[...] * zb
    ht = ht_ref[...] * z

    def gate(g):
        return (jnp.dot(hb, W_ref[g], preferred_element_type=jnp.float32)
                + jnp.dot(h, R_ref[g], preferred_element_type=jnp.float32)
                + jnp.dot(ht, U_ref[g], preferred_element_type=jnp.float32)
                + b_ref[g])

    i = jax.nn.sigmoid(gate(0))
    g_ = jnp.tanh(gate(1))
    o = jax.nn.sigmoid(gate(2))
    f = jax.nn.sigmoid(gate(3))

    ig = i * g_
    c = c_ref[...]

    flush_m = z == 1.0
    update_m = jnp.logical_and(z == 0.0, zb == 1.0)
    copy_m = jnp.logical_not(jnp.logical_or(flush_m, update_m))
    c_act = jnp.where(flush_m, ig, c * f + ig)
    h_act = jnp.tanh(c_act) * o
    h_new = jnp.where(copy_m, h, h_act)
    c_new = jnp.where(copy_m, c, c_act)

    szarg = (jnp.sum(hb * wz_ref[...], axis=1, keepdims=True)
             + jnp.sum(h_new * rz_ref[...], axis=1, keepdims=True)
             + jnp.sum(ht * uz_ref[...], axis=1, keepdims=True)
             + bz_ref[0])
    sz = jax.nn.sigmoid(szarg)
    z_tilde = jnp.clip((sz + 1.0) * 0.5, 0.0, 1.0)
    z_new = jnp.where(z_tilde > 0.5, 1.0, 0.0)

    hout_ref[...] = h_new
    cout_ref[...] = c_new
    zout_ref[pl.ds(base, BT), :] = z_new


def kernel(h, c, h_bottom, h_top, z, z_bottom, W, Wz, R, Rz, U, Uz, b, bz):
    B, H = h.shape
    BT = min(2048, B)
    grid = (B // BT,)

    wz = Wz.reshape(1, -1)
    rz = Rz.reshape(1, -1)
    uz = Uz.reshape(1, -1)

    row = pl.BlockSpec((BT, H), lambda i: (i, 0))
    col = pl.BlockSpec((BT, 1), lambda i: (i, 0))
    full = lambda a: pl.BlockSpec(a.shape, lambda i: (0,) * a.ndim)
    smem = pl.BlockSpec(memory_space=pltpu.SMEM)

    out = pl.pallas_call(
        _cell_kernel,
        grid=grid,
        in_specs=[row, row, row, row, full(z), full(z_bottom),
                  full(W), full(R), full(U), full(b),
                  full(wz), full(rz), full(uz), smem],
        out_specs=[row, row, pl.BlockSpec((B, 1), lambda i: (0, 0))],
        out_shape=[jax.ShapeDtypeStruct((B, H), jnp.float32),
                   jax.ShapeDtypeStruct((B, H), jnp.float32),
                   jax.ShapeDtypeStruct((B, 1), jnp.float32)],
    )(h, c, h_bottom, h_top, z, z_bottom,
      W, R, U, b, wz, rz, uz, bz)
    return (out[0], out[1], out[2])


# speedup vs baseline: 1.0000x; 1.0000x over previous
import jax
import jax.numpy as jnp
from jax.experimental import pallas as pl
from jax.experimental.pallas import tpu as pltpu


def _cell_kernel(h_ref, c_ref, hb_ref, ht_ref, z_ref, zb_ref,
                 W_ref, R_ref, U_ref, b_ref, wz_ref, rz_ref, uz_ref,
                 bz_ref,
                 hout_ref, cout_ref, zout_ref):
    z = z_ref[...]
    zb = zb_ref[...]
    h = h_ref[...]
    hb = hb_ref[...] * zb
    ht = ht_ref[...] * z

    def gate(g):
        return (jnp.dot(hb, W_ref[g], preferred_element_type=jnp.float32)
                + jnp.dot(h, R_ref[g], preferred_element_type=jnp.float32)
                + jnp.dot(ht, U_ref[g], preferred_element_type=jnp.float32)
                + b_ref[g])

    i = jax.nn.sigmoid(gate(0))
    g_ = jnp.tanh(gate(1))
    o = jax.nn.sigmoid(gate(2))
    f = jax.nn.sigmoid(gate(3))

    ig = i * g_
    c = c_ref[...]

    flush_m = z == 1.0
    update_m = jnp.logical_and(z == 0.0, zb == 1.0)
    copy_m = jnp.logical_not(jnp.logical_or(flush_m, update_m))
    c_act = jnp.where(flush_m, ig, c * f + ig)
    h_act = jnp.tanh(c_act) * o
    h_new = jnp.where(copy_m, h, h_act)
    c_new = jnp.where(copy_m, c, c_act)

    szarg = (jnp.sum(hb * wz_ref[...], axis=1, keepdims=True)
             + jnp.sum(h_new * rz_ref[...], axis=1, keepdims=True)
             + jnp.sum(ht * uz_ref[...], axis=1, keepdims=True)
             + bz_ref[0])
    sz = jax.nn.sigmoid(szarg)
    z_tilde = jnp.clip((sz + 1.0) * 0.5, 0.0, 1.0)
    z_new = jnp.where(z_tilde > 0.5, 1.0, 0.0)

    hout_ref[...] = h_new
    cout_ref[...] = c_new
    zout_ref[...] = z_new


def kernel(h, c, h_bottom, h_top, z, z_bottom, W, Wz, R, Rz, U, Uz, b, bz):
    B, H = h.shape
    BT = min(1024, B)
    grid = (B // BT,)

    wz = Wz.reshape(1, -1)
    rz = Rz.reshape(1, -1)
    uz = Uz.reshape(1, -1)

    row = pl.BlockSpec((BT, H), lambda i: (i, 0))
    col = pl.BlockSpec((BT, 1), lambda i: (i, 0))
    full = lambda a: pl.BlockSpec(a.shape, lambda i: (0,) * a.ndim)
    smem = pl.BlockSpec(memory_space=pltpu.SMEM)

    out = pl.pallas_call(
        _cell_kernel,
        grid=grid,
        in_specs=[row, row, row, row, col, col,
                  full(W), full(R), full(U), full(b),
                  full(wz), full(rz), full(uz), smem],
        out_specs=[row, row, col],
        out_shape=[jax.ShapeDtypeStruct((B, H), jnp.float32),
                   jax.ShapeDtypeStruct((B, H), jnp.float32),
                   jax.ShapeDtypeStruct((B, 1), jnp.float32)],
    )(h, c, h_bottom, h_top, z, z_bottom,
      W, R, U, b, wz, rz, uz, bz)
    return (out[0], out[1], out[2])


# speedup vs baseline: 1.0151x; 1.0151x over previous
import jax
import jax.numpy as jnp
from jax.experimental import pallas as pl
from jax.experimental.pallas import tpu as pltpu


def _cell_kernel(h_ref, c_ref, hb_ref, ht_ref, z_ref, zb_ref,
                 W_ref, R_ref, U_ref, b_ref, wz_ref, rz_ref, uz_ref,
                 bz_ref,
                 hout_ref, cout_ref, zout_ref,
                 x_s, w_s):
    z = z_ref[...]
    zb = zb_ref[...]
    h = h_ref[...]
    hb = hb_ref[...] * zb
    ht = ht_ref[...] * z
    H = h.shape[1]

    @pl.when(pl.program_id(0) == 0)
    def _():
        for g in range(4):
            w_s[g, 0:H] = W_ref[g].astype(jnp.bfloat16)
            w_s[g, H:2 * H] = R_ref[g].astype(jnp.bfloat16)
            w_s[g, 2 * H:3 * H] = U_ref[g].astype(jnp.bfloat16)

    x_s[:, 0:H] = hb.astype(jnp.bfloat16)
    x_s[:, H:2 * H] = h.astype(jnp.bfloat16)
    x_s[:, 2 * H:3 * H] = ht.astype(jnp.bfloat16)
    x = x_s[...]

    def gate(g):
        return (jnp.dot(x, w_s[g], preferred_element_type=jnp.float32)
                + b_ref[g])

    i = jax.nn.sigmoid(gate(0))
    g_ = jnp.tanh(gate(1))
    o = jax.nn.sigmoid(gate(2))
    f = jax.nn.sigmoid(gate(3))

    ig = i * g_
    c = c_ref[...]

    flush_m = z == 1.0
    update_m = jnp.logical_and(z == 0.0, zb == 1.0)
    copy_m = jnp.logical_not(jnp.logical_or(flush_m, update_m))
    c_act = jnp.where(flush_m, ig, c * f + ig)
    h_act = jnp.tanh(c_act) * o
    h_new = jnp.where(copy_m, h, h_act)
    c_new = jnp.where(copy_m, c, c_act)

    szarg = (jnp.sum(hb * wz_ref[...], axis=1, keepdims=True)
             + jnp.sum(h_new * rz_ref[...], axis=1, keepdims=True)
             + jnp.sum(ht * uz_ref[...], axis=1, keepdims=True)
             + bz_ref[0])
    sz = jax.nn.sigmoid(szarg)
    z_tilde = jnp.clip((sz + 1.0) * 0.5, 0.0, 1.0)
    z_new = jnp.where(z_tilde > 0.5, 1.0, 0.0)

    hout_ref[...] = h_new
    cout_ref[...] = c_new
    zout_ref[...] = z_new


def kernel(h, c, h_bottom, h_top, z, z_bottom, W, Wz, R, Rz, U, Uz, b, bz):
    B, H = h.shape
    BT = min(2048, B)
    grid = (B // BT,)

    wz = Wz.reshape(1, -1)
    rz = Rz.reshape(1, -1)
    uz = Uz.reshape(1, -1)

    row = pl.BlockSpec((BT, H), lambda i: (i, 0))
    col = pl.BlockSpec((BT, 1), lambda i: (i, 0))
    full = lambda a: pl.BlockSpec(a.shape, lambda i: (0,) * a.ndim)
    smem = pl.BlockSpec(memory_space=pltpu.SMEM)

    out = pl.pallas_call(
        _cell_kernel,
        grid=grid,
        in_specs=[row, row, row, row, col, col,
                  full(W), full(R), full(U), full(b),
                  full(wz), full(rz), full(uz), smem],
        out_specs=[row, row, col],
        out_shape=[jax.ShapeDtypeStruct((B, H), jnp.float32),
                   jax.ShapeDtypeStruct((B, H), jnp.float32),
                   jax.ShapeDtypeStruct((B, 1), jnp.float32)],
        scratch_shapes=[pltpu.VMEM((BT, 3 * H), jnp.bfloat16),
                        pltpu.VMEM((4, 3 * H, H), jnp.bfloat16)],
    )(h, c, h_bottom, h_top, z, z_bottom,
      W, R, U, b, wz, rz, uz, bz)
    return (out[0], out[1], out[2])


# speedup vs baseline: 1.0766x; 1.0606x over previous
import jax
import jax.numpy as jnp
from jax.experimental import pallas as pl
from jax.experimental.pallas import tpu as pltpu


def _cell_kernel(h_ref, c_ref, hb_ref, ht_ref, z_ref, zb_ref,
                 W_ref, R_ref, U_ref, b_ref, wz_ref, rz_ref, uz_ref,
                 bz_ref,
                 hout_ref, cout_ref, zout_ref):
    z = z_ref[...]
    zb = zb_ref[...]
    h = h_ref[...]
    hb = hb_ref[...] * zb
    ht = ht_ref[...] * z

    def gate(g):
        return (jnp.dot(hb, W_ref[g], preferred_element_type=jnp.float32)
                + jnp.dot(h, R_ref[g], preferred_element_type=jnp.float32)
                + jnp.dot(ht, U_ref[g], preferred_element_type=jnp.float32)
                + b_ref[g])

    i = jax.nn.sigmoid(gate(0))
    g_ = jnp.tanh(gate(1))
    o = jax.nn.sigmoid(gate(2))
    f = jax.nn.sigmoid(gate(3))

    ig = i * g_
    c = c_ref[...]

    flush_m = z == 1.0
    update_m = jnp.logical_and(z == 0.0, zb == 1.0)
    copy_m = jnp.logical_not(jnp.logical_or(flush_m, update_m))
    c_act = jnp.where(flush_m, ig, c * f + ig)
    h_act = jnp.tanh(c_act) * o
    h_new = jnp.where(copy_m, h, h_act)
    c_new = jnp.where(copy_m, c, c_act)

    szarg = (jnp.sum(hb * wz_ref[...], axis=1, keepdims=True)
             + jnp.sum(h_new * rz_ref[...], axis=1, keepdims=True)
             + jnp.sum(ht * uz_ref[...], axis=1, keepdims=True)
             + bz_ref[0])
    sz = jax.nn.sigmoid(szarg)
    z_tilde = jnp.clip((sz + 1.0) * 0.5, 0.0, 1.0)
    z_new = jnp.where(z_tilde > 0.5, 1.0, 0.0)

    hout_ref[...] = h_new
    cout_ref[...] = c_new
    zout_ref[...] = z_new


def kernel(h, c, h_bottom, h_top, z, z_bottom, W, Wz, R, Rz, U, Uz, b, bz):
    B, H = h.shape
    BT = min(2048, B)
    grid = (B // BT,)

    wz = Wz.reshape(1, -1)
    rz = Rz.reshape(1, -1)
    uz = Uz.reshape(1, -1)

    row = pl.BlockSpec((BT, H), lambda i: (i, 0))
    col = pl.BlockSpec((BT, 1), lambda i: (i, 0))
    full = lambda a: pl.BlockSpec(a.shape, lambda i: (0,) * a.ndim)
    smem = pl.BlockSpec(memory_space=pltpu.SMEM)

    out = pl.pallas_call(
        _cell_kernel,
        grid=grid,
        in_specs=[row, row, row, row, col, col,
                  full(W), full(R), full(U), full(b),
                  full(wz), full(rz), full(uz), smem],
        out_specs=[row, row, col],
        out_shape=[jax.ShapeDtypeStruct((B, H), jnp.float32),
                   jax.ShapeDtypeStruct((B, H), jnp.float32),
                   jax.ShapeDtypeStruct((B, 1), jnp.float32)],
        compiler_params=pltpu.CompilerParams(
            dimension_semantics=(pltpu.PARALLEL,)),
    )(h, c, h_bottom, h_top, z, z_bottom,
      W, R, U, b, wz, rz, uz, bz)
    return (out[0], out[1], out[2])
